# baseline (speedup 1.0000x reference)
import functools

import jax
import jax.numpy as jnp
from jax import lax
from jax.experimental import pallas as pl
from jax.experimental.pallas import tpu as pltpu
from jax.experimental.pallas import tpu_sc as plsc

_NBINS = 32
_WIDTH = 255.0 / 32.0
_H = 4096
_W = 4096
_N = _H * _W
_NWORK = 32
_PER_W = _N // _NWORK
_CROWS = 8
_WROWS = _H // _NWORK
_NCHUNK = _WROWS // _CROWS
_LANES = 16
_UNROLL = 8
_VPR = _W // _LANES
_ROW = 128


def _sc_hist_body(x_hbm, out_hbm, buf0, buf1, buf2, row_v, sum_v,
                  sem0, sem1, sem2, *tables):
  cid = lax.axis_index("c")
  sid = lax.axis_index("s")
  wid = sid * 2 + cid
  base = wid * _WROWS

  lane = lax.iota(jnp.int32, 16)
  zeros = jnp.zeros((_LANES,), jnp.float32)
  ones = jnp.full((_LANES,), 1.0, jnp.float32)
  addend = (lane * 65 + 16).astype(jnp.float32)
  hi_u = plsc.bitcast(lane * 65 + 48, jnp.uint32)

  sum_v[...] = zeros
  def zbody(i, carry):
    for t in tables:
      t[pl.ds(i * 16, 16)] = zeros
    return carry
  lax.fori_loop(0, 66, zbody, 0)

  bufs = (buf0, buf1, buf2)
  sems = (sem0, sem1, sem2)
  nbuf = len(bufs)

  def chunk_loop(buf):
    def body(i):
      off = i * _LANES
      vs = [buf[u, pl.ds(off, _LANES)] for u in range(_UNROLL)]
      for u in range(_UNROLL):
        q = vs[u] / _WIDTH + addend
        qi = plsc.bitcast(
            jnp.minimum(plsc.bitcast(q.astype(jnp.int32), jnp.uint32), hi_u),
            jnp.int32)
        plsc.addupdate_scatter(tables[u], [qi], ones)
      for u in range(0, _UNROLL, 2):
        plsc.addupdate(sum_v.at[pl.ds(0, _LANES)], vs[u] + vs[u + 1])

    plsc.parallel_loop(0, _VPR, unroll=2)(body)

  descs = {}
  for c in range(nbuf - 1):
    descs[c] = pltpu.async_copy(
        x_hbm.at[pl.ds(base + c * _CROWS, _CROWS)], bufs[c % nbuf],
        sems[c % nbuf])
  for c in range(_NCHUNK):
    n = c + nbuf - 1
    if n < _NCHUNK:
      descs[n] = pltpu.async_copy(
          x_hbm.at[pl.ds(base + n * _CROWS, _CROWS)], bufs[n % nbuf],
          sems[n % nbuf])
    descs.pop(c).wait()
    chunk_loop(bufs[c % nbuf])

  total = sum_v[...]

  r_lo = zeros
  r_hi = zeros
  for t in tables:
    for l in range(16):
      r_lo = r_lo + plsc.load_gather(t, [lane + (65 * l + 16)])
      r_hi = r_hi + plsc.load_gather(t, [lane + (65 * l + 32)])
  row_v[pl.ds(0, 16)] = r_lo
  row_v[pl.ds(16, 16)] = r_hi
  row_v[pl.ds(32, 16)] = jnp.full((_LANES,), jnp.sum(total))
  for j in range(3, 8):
    row_v[pl.ds(j * 16, 16)] = zeros
  pltpu.sync_copy(row_v, out_hbm.at[wid])


def _sc_hist(x2d):
  mesh = plsc.VectorSubcoreMesh(core_axis_name="c", subcore_axis_name="s")
  return pl.kernel(
      _sc_hist_body,
      out_type=jax.ShapeDtypeStruct((_NWORK, _ROW), jnp.float32),
      mesh=mesh,
      scratch_types=[
          pltpu.VMEM((_CROWS, _W), jnp.float32),
          pltpu.VMEM((_CROWS, _W), jnp.float32),
          pltpu.VMEM((_CROWS, _W), jnp.float32),
          pltpu.VMEM((_ROW,), jnp.float32),
          pltpu.VMEM((_LANES,), jnp.float32),
          pltpu.SemaphoreType.DMA,
          pltpu.SemaphoreType.DMA,
          pltpu.SemaphoreType.DMA,
      ] + [pltpu.VMEM((66 * _LANES,), jnp.float32) for _ in range(_UNROLL)],
      compiler_params=pltpu.CompilerParams(needs_layout_passes=False),
  )(x2d)


def _tc_map_body(parts_ref, x_ref, o_ref):
  parts = parts_ref[...]
  hist = jnp.sum(parts[:, :_NBINS], axis=0, keepdims=True)
  mx = jnp.max(hist)
  iota = lax.broadcasted_iota(jnp.int32, (1, _NBINS), 1)
  dom = jnp.min(jnp.where(hist == mx, iota, _NBINS + 1)).astype(jnp.float32)
  mc = jnp.sum(parts[:, _NBINS:_NBINS + 1]) * (1.0 / float(_N))
  xb = x_ref[...]
  pb = jnp.floor(xb * 0.125)
  o_ref[...] = xb * jnp.where(pb == dom, 0.0, mc)


def _tc_map(parts, x2d, block_rows=512):
  grid = (_H // block_rows,)
  return pl.pallas_call(
      _tc_map_body,
      grid=grid,
      in_specs=[
          pl.BlockSpec((_NWORK, _ROW), lambda i: (0, 0)),
          pl.BlockSpec((block_rows, _W), lambda i: (i, 0)),
      ],
      out_specs=pl.BlockSpec((block_rows, _W), lambda i: (i, 0)),
      out_shape=jax.ShapeDtypeStruct((_H, _W), jnp.float32),
  )(parts, x2d)


@jax.jit
def kernel(x):
  B, C, H, W = x.shape
  x2d = x.reshape(H, W)
  parts = _sc_hist(x2d)
  out = _tc_map(parts, x2d)
  return out.reshape(B, C, H, W)


# speedup vs baseline: 1.0092x; 1.0092x over previous
import functools

import jax
import jax.numpy as jnp
from jax import lax
from jax.experimental import pallas as pl
from jax.experimental.pallas import tpu as pltpu
from jax.experimental.pallas import tpu_sc as plsc

_NBINS = 32
_WIDTH = 255.0 / 32.0
_H = 4096
_W = 4096
_N = _H * _W
_NWORK = 32
_PER_W = _N // _NWORK
_CROWS = 8
_WROWS = _H // _NWORK
_NCHUNK = _WROWS // _CROWS
_LANES = 16
_UNROLL = 8
_VPR = _W // _LANES
_ROW = 128


def _sc_hist_body(x_hbm, out_hbm, buf0, buf1, row_v, sum_v,
                  sem0, sem1, *tables):
  cid = lax.axis_index("c")
  sid = lax.axis_index("s")
  wid = sid * 2 + cid
  base = wid * _WROWS

  lane = lax.iota(jnp.int32, 16)
  zeros = jnp.zeros((_LANES,), jnp.float32)
  ones = jnp.full((_LANES,), 1.0, jnp.float32)
  addend = (lane * 65 + 16).astype(jnp.float32)
  hi_u = plsc.bitcast(lane * 65 + 48, jnp.uint32)

  sum_v[...] = zeros
  def zbody(i, carry):
    for t in tables:
      t[pl.ds(i * 16, 16)] = zeros
    return carry
  lax.fori_loop(0, 66, zbody, 0)

  bufs = (buf0, buf1)
  sems = (sem0, sem1)
  nbuf = len(bufs)

  def chunk_loop(buf):
    def body(i):
      off = i * _LANES
      vs = [buf[u, pl.ds(off, _LANES)] for u in range(_UNROLL)]
      for u in range(_UNROLL):
        q = vs[u] / _WIDTH + addend
        qi = plsc.bitcast(
            jnp.minimum(plsc.bitcast(q.astype(jnp.int32), jnp.uint32), hi_u),
            jnp.int32)
        plsc.addupdate_scatter(tables[u], [qi], ones)
      for u in range(0, _UNROLL, 2):
        plsc.addupdate(sum_v.at[pl.ds(0, _LANES)], vs[u] + vs[u + 1])

    plsc.parallel_loop(0, _VPR, unroll=2)(body)

  descs = {}
  for c in range(nbuf - 1):
    descs[c] = pltpu.async_copy(
        x_hbm.at[pl.ds(base + c * _CROWS, _CROWS)], bufs[c % nbuf],
        sems[c % nbuf])
  for c in range(_NCHUNK):
    n = c + nbuf - 1
    if n < _NCHUNK:
      descs[n] = pltpu.async_copy(
          x_hbm.at[pl.ds(base + n * _CROWS, _CROWS)], bufs[n % nbuf],
          sems[n % nbuf])
    descs.pop(c).wait()
    chunk_loop(bufs[c % nbuf])

  total = sum_v[...]

  r_lo = zeros
  r_hi = zeros
  for t in tables:
    for l in range(16):
      r_lo = r_lo + plsc.load_gather(t, [lane + (65 * l + 16)])
      r_hi = r_hi + plsc.load_gather(t, [lane + (65 * l + 32)])
  row_v[pl.ds(0, 16)] = r_lo
  row_v[pl.ds(16, 16)] = r_hi
  row_v[pl.ds(32, 16)] = jnp.full((_LANES,), jnp.sum(total))
  for j in range(3, 8):
    row_v[pl.ds(j * 16, 16)] = zeros
  pltpu.sync_copy(row_v, out_hbm.at[wid])


def _sc_hist(x2d):
  mesh = plsc.VectorSubcoreMesh(core_axis_name="c", subcore_axis_name="s")
  return pl.kernel(
      _sc_hist_body,
      out_type=jax.ShapeDtypeStruct((_NWORK, _ROW), jnp.float32),
      mesh=mesh,
      scratch_types=[
          pltpu.VMEM((_CROWS, _W), jnp.float32),
          pltpu.VMEM((_CROWS, _W), jnp.float32),
          pltpu.VMEM((_ROW,), jnp.float32),
          pltpu.VMEM((_LANES,), jnp.float32),
          pltpu.SemaphoreType.DMA,
          pltpu.SemaphoreType.DMA,
      ] + [pltpu.VMEM((66 * _LANES,), jnp.float32) for _ in range(_UNROLL)],
      compiler_params=pltpu.CompilerParams(needs_layout_passes=False),
  )(x2d)


def _tc_map_body(parts_ref, x_ref, o_ref):
  parts = parts_ref[...]
  hist = jnp.sum(parts[:, :_NBINS], axis=0, keepdims=True)
  mx = jnp.max(hist)
  iota = lax.broadcasted_iota(jnp.int32, (1, _NBINS), 1)
  dom = jnp.min(jnp.where(hist == mx, iota, _NBINS + 1)).astype(jnp.float32)
  mc = jnp.sum(parts[:, _NBINS:_NBINS + 1]) * (1.0 / float(_N))
  xb = x_ref[...]
  pb = jnp.floor(xb * 0.125)
  o_ref[...] = xb * jnp.where(pb == dom, 0.0, mc)


def _tc_map(parts, x2d, block_rows=512):
  grid = (_H // block_rows,)
  return pl.pallas_call(
      _tc_map_body,
      grid=grid,
      in_specs=[
          pl.BlockSpec((_NWORK, _ROW), lambda i: (0, 0)),
          pl.BlockSpec((block_rows, _W), lambda i: (i, 0)),
      ],
      out_specs=pl.BlockSpec((block_rows, _W), lambda i: (i, 0)),
      out_shape=jax.ShapeDtypeStruct((_H, _W), jnp.float32),
  )(parts, x2d)


@jax.jit
def kernel(x):
  B, C, H, W = x.shape
  x2d = x.reshape(H, W)
  parts = _sc_hist(x2d)
  out = _tc_map(parts, x2d)
  return out.reshape(B, C, H, W)


# speedup vs baseline: 1.0094x; 1.0002x over previous
import jax
import jax.numpy as jnp
from jax import lax
from jax.experimental import pallas as pl
from jax.experimental.pallas import tpu as pltpu
from jax.experimental.pallas import tpu_sc as plsc

_NBINS = 32
_WIDTH = 255.0 / 32.0
_H = 4096
_W = 4096
_N = _H * _W
_NWORK = 32
_PER_W = _N // _NWORK
_CROWS = 8
_WROWS = _H // _NWORK
_NCHUNK = _WROWS // _CROWS
_LANES = 16
_UNROLL = 8
_VPR = _W // _LANES
_ROW = 128


def _sc_hist_body(x_hbm, out_hbm, buf0, buf1, row_v, sum_v,
                  sem0, sem1, *tables):
  cid = lax.axis_index("c")
  sid = lax.axis_index("s")
  wid = sid * 2 + cid
  base = wid * _WROWS

  lane = lax.iota(jnp.int32, 16)
  zeros = jnp.zeros((_LANES,), jnp.float32)
  ones = jnp.full((_LANES,), 1.0, jnp.float32)
  addend = (lane * 65 + 16).astype(jnp.float32)
  hi_u = plsc.bitcast(lane * 65 + 48, jnp.uint32)

  sum_v[...] = zeros
  def zbody(i, carry):
    for t in tables:
      t[pl.ds(i * 16, 16)] = zeros
    return carry
  lax.fori_loop(0, 66, zbody, 0)

  bufs = (buf0, buf1)
  sems = (sem0, sem1)
  nbuf = len(bufs)

  def chunk_loop(buf):
    def body(i):
      off = i * _LANES
      vs = [buf[u, pl.ds(off, _LANES)] for u in range(_UNROLL)]
      for u in range(_UNROLL):
        q = vs[u] / _WIDTH + addend
        qi = plsc.bitcast(
            jnp.minimum(plsc.bitcast(q.astype(jnp.int32), jnp.uint32), hi_u),
            jnp.int32)
        plsc.addupdate_scatter(tables[u], [qi], ones)
      for u in range(0, _UNROLL, 2):
        plsc.addupdate(sum_v.at[pl.ds(0, _LANES)], vs[u] + vs[u + 1])

    plsc.parallel_loop(0, _VPR, unroll=2)(body)

  descs = {}
  for c in range(nbuf - 1):
    descs[c] = pltpu.async_copy(
        x_hbm.at[pl.ds(base + c * _CROWS, _CROWS)], bufs[c % nbuf],
        sems[c % nbuf])
  for c in range(_NCHUNK):
    n = c + nbuf - 1
    if n < _NCHUNK:
      descs[n] = pltpu.async_copy(
          x_hbm.at[pl.ds(base + n * _CROWS, _CROWS)], bufs[n % nbuf],
          sems[n % nbuf])
    descs.pop(c).wait()
    chunk_loop(bufs[c % nbuf])

  total = sum_v[...]

  r_lo = zeros
  r_hi = zeros
  for t in tables:
    for l in range(16):
      r_lo = r_lo + plsc.load_gather(t, [lane + (65 * l + 16)])
      r_hi = r_hi + plsc.load_gather(t, [lane + (65 * l + 32)])
  row_v[pl.ds(0, 16)] = r_lo
  row_v[pl.ds(16, 16)] = r_hi
  row_v[pl.ds(32, 16)] = jnp.full((_LANES,), jnp.sum(total))
  for j in range(3, 8):
    row_v[pl.ds(j * 16, 16)] = zeros
  pltpu.sync_copy(row_v, out_hbm.at[wid])


def _sc_hist(x2d):
  mesh = plsc.VectorSubcoreMesh(core_axis_name="c", subcore_axis_name="s")
  return pl.kernel(
      _sc_hist_body,
      out_type=jax.ShapeDtypeStruct((_NWORK, _ROW), jnp.float32),
      mesh=mesh,
      scratch_types=[
          pltpu.VMEM((_CROWS, _W), jnp.float32),
          pltpu.VMEM((_CROWS, _W), jnp.float32),
          pltpu.VMEM((_ROW,), jnp.float32),
          pltpu.VMEM((_LANES,), jnp.float32),
          pltpu.SemaphoreType.DMA,
          pltpu.SemaphoreType.DMA,
      ] + [pltpu.VMEM((66 * _LANES,), jnp.float32) for _ in range(_UNROLL)],
      compiler_params=pltpu.CompilerParams(needs_layout_passes=False),
  )(x2d)


def _tc_map_body(parts_ref, x_ref, o_ref):
  parts = parts_ref[...]
  hist = jnp.sum(parts[:, :_NBINS], axis=0, keepdims=True)
  mx = jnp.max(hist)
  iota = lax.broadcasted_iota(jnp.int32, (1, _NBINS), 1)
  dom = jnp.min(jnp.where(hist == mx, iota, _NBINS + 1)).astype(jnp.float32)
  mc = jnp.sum(parts[:, _NBINS:_NBINS + 1]) * (1.0 / float(_N))
  xb = x_ref[...]
  pb = jnp.floor(xb * 0.125)
  o_ref[...] = xb * jnp.where(pb == dom, 0.0, mc)


def _tc_map(parts, x2d, block_rows=512):
  grid = (_H // block_rows,)
  return pl.pallas_call(
      _tc_map_body,
      grid=grid,
      in_specs=[
          pl.BlockSpec((_NWORK, _ROW), lambda i: (0, 0)),
          pl.BlockSpec((block_rows, _W), lambda i: (i, 0)),
      ],
      out_specs=pl.BlockSpec((block_rows, _W), lambda i: (i, 0)),
      out_shape=jax.ShapeDtypeStruct((_H, _W), jnp.float32),
  )(parts, x2d)


@jax.jit
def kernel(x):
  B, C, H, W = x.shape
  x2d = x.reshape(H, W)
  parts = _sc_hist(x2d)
  out = _tc_map(parts, x2d)
  return out.reshape(B, C, H, W)
